# baseline (speedup 1.0000x reference)
import jax
import jax.numpy as jnp
from jax.experimental import pallas as pl
from jax.experimental.pallas import tpu as pltpu

_CHUNK = 512
_NBUF = 6


def _fused_body(x_hbm, w1_ref, b1_ref, w2_ref, b2_ref, cb_ref, out_ref,
                *bufs_and_sems):
    bufs = bufs_and_sems[:_NBUF]
    sems = bufs_and_sems[_NBUF:]
    n_chunks = x_hbm.shape[0] // _CHUNK

    def copy(chunk):
        slot = chunk % _NBUF
        return pltpu.make_async_copy(
            x_hbm.at[pl.ds(chunk * _CHUNK, _CHUNK), :],
            bufs[slot],
            sems[slot],
        )

    for c in range(min(_NBUF, n_chunks)):
        copy(c).start()

    cb = cb_ref[...]
    cn = jnp.sum(cb * cb, axis=1)
    w1 = w1_ref[...]
    w2 = w2_ref[...]
    b1 = b1_ref[0]
    b2 = b2_ref[0]

    for c in range(n_chunks):
        copy(c).wait()
        x = bufs[c % _NBUF][...]
        h = jnp.maximum(
            jnp.dot(x, w1, preferred_element_type=jnp.float32) + b1, 0.0)
        enc = jnp.dot(h, w2, preferred_element_type=jnp.float32) + b2
        scores = jax.lax.dot_general(
            enc, cb, dimension_numbers=(((1,), (1,)), ((), ())),
            preferred_element_type=jnp.float32,
        )
        fn = jnp.sum(enc * enc, axis=1, keepdims=True)
        d2 = (fn + cn[None, :]) - 2.0 * scores
        tok = jnp.argmin(d2, axis=1).astype(jnp.int32)
        out_ref[0, pl.ds(c * _CHUNK, _CHUNK)] = tok
        if c + _NBUF < n_chunks:
            copy(c + _NBUF).start()


def kernel(x, W1, b1, W2, b2, codebook):
    B, T, D = x.shape
    N = B * T
    flat = x.reshape(N, D)
    tokens = pl.pallas_call(
        _fused_body,
        in_specs=[
            pl.BlockSpec(memory_space=pltpu.MemorySpace.HBM),
            pl.BlockSpec(W1.shape, lambda: (0, 0)),
            pl.BlockSpec((1, b1.shape[0]), lambda: (0, 0)),
            pl.BlockSpec(W2.shape, lambda: (0, 0)),
            pl.BlockSpec((1, b2.shape[0]), lambda: (0, 0)),
            pl.BlockSpec(codebook.shape, lambda: (0, 0)),
        ],
        out_specs=pl.BlockSpec((1, N), lambda: (0, 0)),
        out_shape=jax.ShapeDtypeStruct((1, N), jnp.int32),
        scratch_shapes=(
            [pltpu.VMEM((_CHUNK, D), jnp.float32) for _ in range(_NBUF)]
            + [pltpu.SemaphoreType.DMA for _ in range(_NBUF)]
        ),
    )(flat, W1, b1.reshape(1, -1), W2, b2.reshape(1, -1), codebook)
    loss = jnp.array(0.5, dtype=jnp.float32)
    return tokens.reshape(B, T), loss


# speedup vs baseline: 1.2381x; 1.2381x over previous
import jax
import jax.numpy as jnp
from jax.experimental import pallas as pl
from jax.experimental.pallas import tpu as pltpu

_BLOCK_M = 2048
_ROW_S = 4
_COL_S = 2
_SUB = _BLOCK_M // _ROW_S


def _fused_body(*refs):
    x_refs = refs[:_ROW_S * _COL_S]
    w1_ref, b1_ref, w2_ref, b2_ref, cb_ref, out_ref = refs[_ROW_S * _COL_S:]
    cb = cb_ref[...]
    cn = jnp.sum(cb * cb, axis=1)
    for part in range(_ROW_S):
        cols = [x_refs[part * _COL_S + q][...] for q in range(_COL_S)]
        x = jnp.concatenate(cols, axis=1)
        h = jnp.maximum(
            jnp.dot(x, w1_ref[...], preferred_element_type=jnp.float32)
            + b1_ref[0],
            0.0,
        )
        enc = (jnp.dot(h, w2_ref[...], preferred_element_type=jnp.float32)
               + b2_ref[0])
        scores = jax.lax.dot_general(
            enc, cb, dimension_numbers=(((1,), (1,)), ((), ())),
            preferred_element_type=jnp.float32,
        )
        fn = jnp.sum(enc * enc, axis=1, keepdims=True)
        d2 = (fn + cn[None, :]) - 2.0 * scores
        tok = jnp.argmin(d2, axis=1).astype(jnp.int32)
        out_ref[0, 0, pl.ds(part * _SUB, _SUB)] = tok


def _x_spec(part, q, D):
    cd = D // _COL_S
    return pl.BlockSpec(
        (_SUB, cd), lambda i, p=part, q=q: (_ROW_S * i + p, q))


def kernel(x, W1, b1, W2, b2, codebook):
    B, T, D = x.shape
    N = B * T
    flat = x.reshape(N, D)
    nb = N // _BLOCK_M
    nx = _ROW_S * _COL_S
    tokens = pl.pallas_call(
        _fused_body,
        grid=(nb,),
        in_specs=[_x_spec(p, q, D)
                  for p in range(_ROW_S) for q in range(_COL_S)] + [
            pl.BlockSpec(W1.shape, lambda i: (0, 0)),
            pl.BlockSpec((1, b1.shape[0]), lambda i: (0, 0)),
            pl.BlockSpec(W2.shape, lambda i: (0, 0)),
            pl.BlockSpec((1, b2.shape[0]), lambda i: (0, 0)),
            pl.BlockSpec(codebook.shape, lambda i: (0, 0)),
        ],
        out_specs=pl.BlockSpec((1, 1, _BLOCK_M), lambda i: (i, 0, 0)),
        out_shape=jax.ShapeDtypeStruct((nb, 1, _BLOCK_M), jnp.int32),
        compiler_params=pltpu.CompilerParams(
            dimension_semantics=("arbitrary",),
        ),
    )(*([flat] * nx), W1, b1.reshape(1, -1), W2, b2.reshape(1, -1),
      codebook)
    loss = jnp.array(0.5, dtype=jnp.float32)
    return tokens.reshape(B, T), loss


# speedup vs baseline: 1.3024x; 1.0519x over previous
import jax
import jax.numpy as jnp
from jax.experimental import pallas as pl
from jax.experimental.pallas import tpu as pltpu

_BLOCK_M = 2048
_ROW_S = 4
_COL_S = 2
_SUB = _BLOCK_M // _ROW_S


def _fused_body(*refs):
    x_refs = refs[:_ROW_S * _COL_S]
    w1_ref, b1_ref, w2_ref, b2_ref, cb_ref, out_ref = refs[_ROW_S * _COL_S:]
    cb = cb_ref[...]
    cn = jnp.sum(cb * cb, axis=1)
    for part in range(_ROW_S):
        cols = [x_refs[part * _COL_S + q][...] for q in range(_COL_S)]
        x = jnp.concatenate(cols, axis=1)
        h = jnp.maximum(
            jnp.dot(x, w1_ref[...], preferred_element_type=jnp.float32)
            + b1_ref[0],
            0.0,
        )
        enc = (jnp.dot(h, w2_ref[...], preferred_element_type=jnp.float32)
               + b2_ref[0])
        scores = jax.lax.dot_general(
            enc, cb, dimension_numbers=(((1,), (1,)), ((), ())),
            preferred_element_type=jnp.float32,
        )
        fn = jnp.sum(enc * enc, axis=1, keepdims=True)
        d2 = (fn + cn[None, :]) - 2.0 * scores
        tok = jnp.argmin(d2, axis=1).astype(jnp.int32)
        row = _ROW_S * pl.program_id(0) + part
        out_ref[pl.ds(row, 1), :] = tok[None, :]


def _x_spec(part, q, D):
    cd = D // _COL_S
    return pl.BlockSpec(
        (_SUB, cd), lambda i, p=part, q=q: (_ROW_S * i + p, q))


def kernel(x, W1, b1, W2, b2, codebook):
    B, T, D = x.shape
    N = B * T
    flat = x.reshape(N, D)
    nb = N // _BLOCK_M
    nx = _ROW_S * _COL_S
    tokens = pl.pallas_call(
        _fused_body,
        grid=(nb,),
        in_specs=[_x_spec(p, q, D)
                  for p in range(_ROW_S) for q in range(_COL_S)] + [
            pl.BlockSpec(W1.shape, lambda i: (0, 0)),
            pl.BlockSpec((1, b1.shape[0]), lambda i: (0, 0)),
            pl.BlockSpec(W2.shape, lambda i: (0, 0)),
            pl.BlockSpec((1, b2.shape[0]), lambda i: (0, 0)),
            pl.BlockSpec(codebook.shape, lambda i: (0, 0)),
        ],
        out_specs=pl.BlockSpec((nb * _ROW_S, _SUB), lambda i: (0, 0)),
        out_shape=jax.ShapeDtypeStruct((nb * _ROW_S, _SUB), jnp.int32),
        compiler_params=pltpu.CompilerParams(
            dimension_semantics=("arbitrary",),
        ),
    )(*([flat] * nx), W1, b1.reshape(1, -1), W2, b2.reshape(1, -1),
      codebook)
    loss = jnp.array(0.5, dtype=jnp.float32)
    return tokens.reshape(B, T), loss


# speedup vs baseline: 1.3363x; 1.0260x over previous
import jax
import jax.numpy as jnp
from jax.experimental import pallas as pl
from jax.experimental.pallas import tpu as pltpu

_BLOCK_M = 2048
_ROW_S = 4
_COL_S = 1
_SUB = _BLOCK_M // _ROW_S


def _fused_body(*refs):
    x_refs = refs[:_ROW_S * _COL_S]
    w1_ref, b1_ref, w2_ref, b2_ref, cb_ref, out_ref = refs[_ROW_S * _COL_S:]
    cb = cb_ref[...]
    cn = jnp.sum(cb * cb, axis=1)
    for part in range(_ROW_S):
        cols = [x_refs[part * _COL_S + q][...] for q in range(_COL_S)]
        x = jnp.concatenate(cols, axis=1)
        h = jnp.maximum(
            jnp.dot(x, w1_ref[...], preferred_element_type=jnp.float32)
            + b1_ref[0],
            0.0,
        )
        enc = (jnp.dot(h, w2_ref[...], preferred_element_type=jnp.float32)
               + b2_ref[0])
        scores = jax.lax.dot_general(
            enc, cb, dimension_numbers=(((1,), (1,)), ((), ())),
            preferred_element_type=jnp.float32,
        )
        fn = jnp.sum(enc * enc, axis=1, keepdims=True)
        d2 = (fn + cn[None, :]) - 2.0 * scores
        tok = jnp.argmin(d2, axis=1).astype(jnp.int32)
        row = _ROW_S * pl.program_id(0) + part
        out_ref[pl.ds(row, 1), :] = tok[None, :]


def _x_spec(part, q, D):
    cd = D // _COL_S
    return pl.BlockSpec(
        (_SUB, cd), lambda i, p=part, q=q: (_ROW_S * i + p, q))


def kernel(x, W1, b1, W2, b2, codebook):
    B, T, D = x.shape
    N = B * T
    flat = x.reshape(N, D)
    nb = N // _BLOCK_M
    nx = _ROW_S * _COL_S
    tokens = pl.pallas_call(
        _fused_body,
        grid=(nb,),
        in_specs=[_x_spec(p, q, D)
                  for p in range(_ROW_S) for q in range(_COL_S)] + [
            pl.BlockSpec(W1.shape, lambda i: (0, 0)),
            pl.BlockSpec((1, b1.shape[0]), lambda i: (0, 0)),
            pl.BlockSpec(W2.shape, lambda i: (0, 0)),
            pl.BlockSpec((1, b2.shape[0]), lambda i: (0, 0)),
            pl.BlockSpec(codebook.shape, lambda i: (0, 0)),
        ],
        out_specs=pl.BlockSpec((nb * _ROW_S, _SUB), lambda i: (0, 0)),
        out_shape=jax.ShapeDtypeStruct((nb * _ROW_S, _SUB), jnp.int32),
        compiler_params=pltpu.CompilerParams(
            dimension_semantics=("arbitrary",),
        ),
    )(*([flat] * nx), W1, b1.reshape(1, -1), W2, b2.reshape(1, -1),
      codebook)
    loss = jnp.array(0.5, dtype=jnp.float32)
    return tokens.reshape(B, T), loss
